# baseline (device time: 104921 ns/iter reference)
import jax
import jax.numpy as jnp
from jax import lax
from jax.experimental import pallas as pl
from jax.experimental.pallas import tpu as pltpu

N_DEV_Y = 2
EPS = 1e-5
TILE = 1024


def kernel(x, gamma, beta):
    m, n_loc = x.shape
    n_glob = n_loc * N_DEV_Y
    t = m // TILE

    def stats_body(x_ref, stats_ref, local_ref, recv_ref, send_sem, recv_sem):
        i = pl.program_id(0)
        my_x = lax.axis_index("x")
        my_y = lax.axis_index("y")

        @pl.when(i == 0)
        def _():
            barrier = pltpu.get_barrier_semaphore()
            pl.semaphore_signal(
                barrier,
                inc=1,
                device_id=(my_x, 1 - my_y),
                device_id_type=pl.DeviceIdType.MESH,
            )
            pl.semaphore_wait(barrier, 1)

        xt = x_ref[...]
        s = jnp.sum(xt, axis=1, keepdims=True)
        sq = jnp.sum(xt * xt, axis=1, keepdims=True)
        local_ref[pl.ds(i * TILE, TILE), :] = jnp.concatenate([s, sq], axis=1)

        @pl.when(i == t - 1)
        def _():
            rdma = pltpu.make_async_remote_copy(
                src_ref=local_ref,
                dst_ref=recv_ref,
                send_sem=send_sem,
                recv_sem=recv_sem,
                device_id=(my_x, 1 - my_y),
                device_id_type=pl.DeviceIdType.MESH,
            )
            rdma.start()
            rdma.wait()
            tot = local_ref[...] + recv_ref[...]
            mean = tot[:, 0:1] * (1.0 / n_glob)
            var = tot[:, 1:2] * (1.0 / n_glob) - mean * mean
            rstd = lax.rsqrt(var + EPS)
            stats_ref[...] = jnp.concatenate([mean, rstd], axis=1)

    stats = pl.pallas_call(
        stats_body,
        grid=(t,),
        in_specs=[pl.BlockSpec((TILE, n_loc), lambda i: (i, 0))],
        out_specs=pl.BlockSpec((m, 2), lambda i: (0, 0)),
        out_shape=jax.ShapeDtypeStruct((m, 2), jnp.float32),
        scratch_shapes=[
            pltpu.VMEM((m, 2), jnp.float32),
            pltpu.VMEM((m, 2), jnp.float32),
            pltpu.SemaphoreType.DMA,
            pltpu.SemaphoreType.DMA,
        ],
        compiler_params=pltpu.CompilerParams(collective_id=0),
    )(x)

    g2 = gamma.reshape(1, n_loc)
    b2 = beta.reshape(1, n_loc)

    def norm_body(x_ref, g_ref, b_ref, st_ref, o_ref):
        xt = x_ref[...]
        mean = st_ref[:, 0:1]
        rstd = st_ref[:, 1:2]
        o_ref[...] = ((xt - mean) * rstd * g_ref[...] + b_ref[...]).astype(
            jnp.bfloat16
        )

    return pl.pallas_call(
        norm_body,
        grid=(t,),
        in_specs=[
            pl.BlockSpec((TILE, n_loc), lambda i: (i, 0)),
            pl.BlockSpec((1, n_loc), lambda i: (0, 0)),
            pl.BlockSpec((1, n_loc), lambda i: (0, 0)),
            pl.BlockSpec((TILE, 2), lambda i: (i, 0)),
        ],
        out_specs=pl.BlockSpec((TILE, n_loc), lambda i: (i, 0)),
        out_shape=jax.ShapeDtypeStruct((m, n_loc), jnp.bfloat16),
    )(x, g2, b2, stats)
